# baseline (speedup 1.0000x reference)
import functools

import jax
import jax.numpy as jnp
from jax import lax
from jax.experimental import pallas as pl
from jax.experimental.pallas import tpu as pltpu
from jax.experimental.pallas import tpu_sc as plsc


_CHUNK_ROWS = 8
_NBUF = 6
_PF = 3


def kernel(embeddings, pos_table):
    Bn, Tn, Hn = embeddings.shape
    info = plsc.get_sparse_core_info()
    nw = info.num_cores * info.num_subcores
    rows_w = Tn // nw
    chunks = rows_w // _CHUNK_ROWS
    steps = chunks * Bn
    colshift = (Hn - 1).bit_length()
    mesh = plsc.VectorSubcoreMesh(core_axis_name="c", subcore_axis_name="s")

    @functools.partial(
        pl.kernel,
        mesh=mesh,
        out_type=jax.ShapeDtypeStruct((Bn, Tn, Hn), jnp.float32),
        scratch_types=[
            pltpu.VMEM((1, _CHUNK_ROWS, Hn), jnp.float32),
            pltpu.VMEM((_NBUF, _CHUNK_ROWS, Hn), jnp.float32),
            pltpu.SemaphoreType.DMA((1,)),
            pltpu.SemaphoreType.DMA((_NBUF,)),
            pltpu.SemaphoreType.DMA((_NBUF,)),
        ],
    )
    def sc_k(emb_hbm, pos_hbm, out_hbm, pos_v, emb_v, psem, lsem, ssem):
        wid = lax.axis_index("s") * info.num_cores + lax.axis_index("c")
        t0 = wid * rows_w

        def load_pos(c):
            return pltpu.async_copy(
                pos_hbm.at[pl.ds(t0 + c * _CHUNK_ROWS, _CHUNK_ROWS)],
                pos_v.at[0], psem.at[0])

        def load_emb(s):
            c, b = divmod(s, Bn)
            return pltpu.async_copy(
                emb_hbm.at[b, pl.ds(t0 + c * _CHUNK_ROWS, _CHUNK_ROWS)],
                emb_v.at[s % _NBUF], lsem.at[s % _NBUF])

        def store_out(s):
            c, b = divmod(s, Bn)
            return pltpu.async_copy(
                emb_v.at[s % _NBUF],
                out_hbm.at[b, pl.ds(t0 + c * _CHUNK_ROWS, _CHUNK_ROWS)],
                ssem.at[s % _NBUF])

        pos_cps = {0: load_pos(0)}
        loads = {s: load_emb(s) for s in range(min(_PF, steps))}
        stores = {}
        for s in range(steps):
            c, b = divmod(s, Bn)
            if b == 0:
                pos_cps.pop(c).wait()
            loads.pop(s).wait()

            @plsc.parallel_loop(0, _CHUNK_ROWS * Hn, step=16, unroll=4)
            def _(i):
                r = i >> colshift
                col = pl.multiple_of(i - (r << colshift), 16)
                pv = pos_v[0, r, pl.ds(col, 16)]
                emb_v[s % _NBUF, r, pl.ds(col, 16)] = (
                    emb_v[s % _NBUF, r, pl.ds(col, 16)] + pv)

            stores[s] = store_out(s)
            if b == Bn - 1 and c + 1 < chunks:
                pos_cps[c + 1] = load_pos(c + 1)
            ns = s + _PF
            if ns < steps:
                if ns - _NBUF >= 0:
                    stores.pop(ns - _NBUF).wait()
                loads[ns] = load_emb(ns)
        for cp in stores.values():
            cp.wait()

    return sc_k(embeddings, pos_table)


# speedup vs baseline: 1.0972x; 1.0972x over previous
import functools

import jax
import jax.numpy as jnp
from jax import lax
from jax.experimental import pallas as pl
from jax.experimental.pallas import tpu as pltpu
from jax.experimental.pallas import tpu_sc as plsc


_CHUNK_ROWS = 4
_NBUF = 8
_SS = 4
_PF = 4


def kernel(embeddings, pos_table):
    Bn, Tn, Hn = embeddings.shape
    info = plsc.get_sparse_core_info()
    nsub = info.num_subcores
    nw = info.num_cores * nsub
    rows_w = Tn // nw
    chunks = rows_w // _CHUNK_ROWS
    steps = chunks * Bn
    colshift = (Hn - 1).bit_length()
    mesh = plsc.VectorSubcoreMesh(core_axis_name="c", subcore_axis_name="s")

    @functools.partial(
        pl.kernel,
        mesh=mesh,
        out_type=jax.ShapeDtypeStruct((Bn, Tn, Hn), jnp.float32),
        scratch_types=[
            pltpu.VMEM((2, _CHUNK_ROWS, Hn), jnp.float32),
            pltpu.VMEM((_NBUF, _CHUNK_ROWS, Hn), jnp.float32),
            pltpu.VMEM_SHARED((nsub, _SS, _CHUNK_ROWS, Hn), jnp.float32),
            pltpu.SemaphoreType.DMA((2,)),
            pltpu.SemaphoreType.DMA((_NBUF,)),
            pltpu.SemaphoreType.DMA((_SS,)),
            pltpu.SemaphoreType.DMA((_SS,)),
        ],
    )
    def sc_k(emb_hbm, pos_hbm, out_hbm, pos_v, emb_v, spm, psem, lsem,
             asem, bsem):
        cid = lax.axis_index("c")
        sid = lax.axis_index("s")
        wid = sid * info.num_cores + cid
        t0 = wid * rows_w

        def load_pos(c):
            return pltpu.async_copy(
                pos_hbm.at[pl.ds(t0 + c * _CHUNK_ROWS, _CHUNK_ROWS)],
                pos_v.at[c % 2], psem.at[c % 2])

        def load_emb(s):
            c, b = divmod(s, Bn)
            return pltpu.async_copy(
                emb_hbm.at[b, pl.ds(t0 + c * _CHUNK_ROWS, _CHUNK_ROWS)],
                emb_v.at[s % _NBUF], lsem.at[s % _NBUF])

        def stage_out(s):
            return pltpu.async_copy(
                emb_v.at[s % _NBUF], spm.at[sid, s % _SS], asem.at[s % _SS])

        def store_out(s):
            c, b = divmod(s, Bn)
            return pltpu.async_copy(
                spm.at[sid, s % _SS],
                out_hbm.at[b, pl.ds(t0 + c * _CHUNK_ROWS, _CHUNK_ROWS)],
                bsem.at[s % _SS])

        pos_cps = {c: load_pos(c) for c in range(min(2, chunks))}
        loads = {s: load_emb(s) for s in range(min(_PF, steps))}
        stages = {}
        stores = {}
        for s in range(steps):
            c, b = divmod(s, Bn)
            if b == 0:
                pos_cps.pop(c).wait()
            loads.pop(s).wait()

            @plsc.parallel_loop(0, _CHUNK_ROWS * Hn, step=16, unroll=4)
            def _(i):
                r = i >> colshift
                col = pl.multiple_of(i - (r << colshift), 16)
                pv = pos_v[c % 2, r, pl.ds(col, 16)]
                emb_v[s % _NBUF, r, pl.ds(col, 16)] = (
                    emb_v[s % _NBUF, r, pl.ds(col, 16)] + pv)

            if s - _SS in stores:
                stores.pop(s - _SS).wait()
            stages[s] = stage_out(s)
            if b == Bn - 1 and c + 2 < chunks:
                pos_cps[c + 2] = load_pos(c + 2)
            if s - 1 in stages:
                stages.pop(s - 1).wait()
                stores[s - 1] = store_out(s - 1)
            ns = s + _PF
            if ns < steps:
                if ns - _NBUF in stages:
                    stages.pop(ns - _NBUF).wait()
                loads[ns] = load_emb(ns)
        for s in sorted(stages):
            stages[s].wait()
            stores[s] = store_out(s)
        for cp in stores.values():
            cp.wait()

    return sc_k(embeddings, pos_table)


# speedup vs baseline: 1.1037x; 1.0059x over previous
import functools

import jax
import jax.numpy as jnp
from jax import lax
from jax.experimental import pallas as pl
from jax.experimental.pallas import tpu as pltpu
from jax.experimental.pallas import tpu_sc as plsc


_CHUNK_ROWS = 4
_NBUF = 12
_PF = 6


def kernel(embeddings, pos_table):
    Bn, Tn, Hn = embeddings.shape
    info = plsc.get_sparse_core_info()
    nw = info.num_cores * info.num_subcores
    rows_w = Tn // nw
    chunks = rows_w // _CHUNK_ROWS
    steps = chunks * Bn
    colshift = (Hn - 1).bit_length()
    mesh = plsc.VectorSubcoreMesh(core_axis_name="c", subcore_axis_name="s")

    @functools.partial(
        pl.kernel,
        mesh=mesh,
        out_type=jax.ShapeDtypeStruct((Bn, Tn, Hn), jnp.float32),
        scratch_types=[
            pltpu.VMEM((2, _CHUNK_ROWS, Hn), jnp.float32),
            pltpu.VMEM((_NBUF, _CHUNK_ROWS, Hn), jnp.float32),
            pltpu.SemaphoreType.DMA((2,)),
            pltpu.SemaphoreType.DMA((_NBUF,)),
            pltpu.SemaphoreType.DMA((_NBUF,)),
        ],
    )
    def sc_k(emb_hbm, pos_hbm, out_hbm, pos_v, emb_v, psem, lsem, ssem):
        wid = lax.axis_index("s") * info.num_cores + lax.axis_index("c")
        t0 = wid * rows_w

        def load_pos(c):
            return pltpu.async_copy(
                pos_hbm.at[pl.ds(t0 + c * _CHUNK_ROWS, _CHUNK_ROWS)],
                pos_v.at[c % 2], psem.at[c % 2])

        def load_emb(s):
            c, b = divmod(s, Bn)
            return pltpu.async_copy(
                emb_hbm.at[b, pl.ds(t0 + c * _CHUNK_ROWS, _CHUNK_ROWS)],
                emb_v.at[s % _NBUF], lsem.at[s % _NBUF])

        def store_out(s):
            c, b = divmod(s, Bn)
            return pltpu.async_copy(
                emb_v.at[s % _NBUF],
                out_hbm.at[b, pl.ds(t0 + c * _CHUNK_ROWS, _CHUNK_ROWS)],
                ssem.at[s % _NBUF])

        pos_cps = {c: load_pos(c) for c in range(min(2, chunks))}
        loads = {s: load_emb(s) for s in range(min(_PF, steps))}
        stores = {}
        for s in range(steps):
            c, b = divmod(s, Bn)
            if b == 0:
                pos_cps.pop(c).wait()
            loads.pop(s).wait()

            @plsc.parallel_loop(0, _CHUNK_ROWS * Hn, step=16, unroll=4)
            def _(i):
                r = i >> colshift
                col = pl.multiple_of(i - (r << colshift), 16)
                pv = pos_v[c % 2, r, pl.ds(col, 16)]
                emb_v[s % _NBUF, r, pl.ds(col, 16)] = (
                    emb_v[s % _NBUF, r, pl.ds(col, 16)] + pv)

            stores[s] = store_out(s)
            if b == Bn - 1 and c + 2 < chunks:
                pos_cps[c + 2] = load_pos(c + 2)
            ns = s + _PF
            if ns < steps:
                if ns - _NBUF >= 0:
                    stores.pop(ns - _NBUF).wait()
                loads[ns] = load_emb(ns)
        for cp in stores.values():
            cp.wait()

    return sc_k(embeddings, pos_table)
